# baseline (speedup 1.0000x reference)
import functools

import jax
import jax.numpy as jnp
from jax.experimental import pallas as pl
from jax.experimental.pallas import tpu as pltpu

_N_PRIORS = 8732
_P_PAD = 9216
_SUB = 8
_LANES = 1152
_N_CLASSES = 21
_N_OBJ = 16
_THRESHOLD = 0.5
_NEG_POS_RATIO = 3
_MINE_ROUNDS = 6
_IMGS_PER_PROG = 2


def _loss_kernel(boxes_ref, labels_ref, priors_ref, locs_ref, scores_ref,
                 out_ref):
    f32 = jnp.float32
    shp = (_SUB, _LANES)

    pcx = priors_ref[0]
    pcy = priors_ref[1]
    pw = priors_ref[2]
    ph = priors_ref[3]
    px0 = pcx - pw * 0.5
    py0 = pcy - ph * 0.5
    px1 = pcx + pw * 0.5
    py1 = pcy + ph * 0.5
    area_p = (px1 - px0) * (py1 - py0)

    iota_p = (jax.lax.broadcasted_iota(jnp.int32, shp, 0) * _LANES
              + jax.lax.broadcasted_iota(jnp.int32, shp, 1))
    valid = iota_p < _N_PRIORS

    def per_image(g):
        best_ov = jnp.full(shp, -1.0, f32)
        lab = jnp.zeros(shp, jnp.int32)
        gx0 = jnp.zeros(shp, f32)
        gy0 = jnp.zeros(shp, f32)
        gx1 = jnp.zeros(shp, f32)
        gy1 = jnp.zeros(shp, f32)
        prior_of_obj = []
        for j in range(_N_OBJ):
            bx0 = boxes_ref[g, 0, j, 0]
            by0 = boxes_ref[g, 0, j, 1]
            bx1 = boxes_ref[g, 0, j, 2]
            by1 = boxes_ref[g, 0, j, 3]
            area_b = (bx1 - bx0) * (by1 - by0)
            wx = jnp.clip(jnp.minimum(bx1, px1) - jnp.maximum(bx0, px0),
                          0.0, None)
            wy = jnp.clip(jnp.minimum(by1, py1) - jnp.maximum(by0, py0),
                          0.0, None)
            inter = wx * wy
            iou = inter / (area_b + area_p - inter)
            mx = jnp.max(iou)
            pj = jnp.min(jnp.where(iou == mx, iota_p, _N_PRIORS))
            prior_of_obj.append(pj)
            upd = iou > best_ov
            best_ov = jnp.where(upd, iou, best_ov)
            lab = jnp.where(upd, labels_ref[g, 0, 0, j], lab)
            gx0 = jnp.where(upd, bx0, gx0)
            gy0 = jnp.where(upd, by0, gy0)
            gx1 = jnp.where(upd, bx1, gx1)
            gy1 = jnp.where(upd, by1, gy1)

        lab = jnp.where(best_ov < _THRESHOLD, 0, lab)
        for j in range(_N_OBJ):
            m = iota_p == prior_of_obj[j]
            lab = jnp.where(m, labels_ref[g, 0, 0, j], lab)
            gx0 = jnp.where(m, boxes_ref[g, 0, j, 0], gx0)
            gy0 = jnp.where(m, boxes_ref[g, 0, j, 1], gy0)
            gx1 = jnp.where(m, boxes_ref[g, 0, j, 2], gx1)
            gy1 = jnp.where(m, boxes_ref[g, 0, j, 3], gy1)
        pos = lab != 0
        posf = pos.astype(f32)
        n_pos = jnp.sum(posf)

        gcx = (gx0 + gx1) * 0.5
        gcy = (gy0 + gy1) * 0.5
        gw = gx1 - gx0
        gh = gy1 - gy0
        t0 = (gcx - pcx) / (pw / 10.0)
        t1 = (gcy - pcy) / (ph / 10.0)
        t2 = jnp.log(gw / pw) * 5.0
        t3 = jnp.log(gh / ph) * 5.0
        sl = jnp.zeros(shp, f32)
        for i, t in enumerate((t0, t1, t2, t3)):
            diff = jnp.abs(locs_ref[g, i] - t)
            sl = sl + jnp.where(diff < 1.0, 0.5 * diff * diff, diff - 0.5)
        loc_loss = jnp.sum(sl * posf)

        m = scores_ref[g, 0]
        for c in range(1, _N_CLASSES):
            m = jnp.maximum(m, scores_ref[g, c])
        ssum = jnp.zeros(shp, f32)
        sal = jnp.zeros(shp, f32)
        for c in range(_N_CLASSES):
            shc = scores_ref[g, c] - m
            ssum = ssum + jnp.exp(shc)
            sal = jnp.where(lab == c, shc, sal)
        conf = jnp.log(ssum) - sal
        conf_pos = jnp.sum(conf * posf)
        neg = jnp.where(pos | jnp.logical_not(valid), 0.0, conf)

        k = n_pos * float(_NEG_POS_RATIO)
        blo = jnp.float32(0.0)
        bhi = jnp.max(neg)
        for _ in range(_MINE_ROUNDS):
            w = (bhi - blo) * 0.125
            cnts = [jnp.sum((neg > (blo + w * float(i))).astype(f32))
                    for i in range(1, 8)]
            s = jnp.float32(0.0)
            for c in cnts:
                s = s + (c > k).astype(f32)
            blo = blo + w * s
            bhi = blo + w
        t = 0.5 * (blo + bhi)
        sel = neg > t
        cnt_t = jnp.sum(sel.astype(f32))
        conf_neg = jnp.sum(jnp.where(sel, neg, 0.0)) + (k - cnt_t) * t

        return (conf_pos + conf_neg + loc_loss) / n_pos

    total = per_image(0)
    for g in range(1, _IMGS_PER_PROG):
        total = total + per_image(g)
    out_ref[0, 0, 0] = total


@jax.jit
def kernel(pred_locs, pred_scores, boxes, labels, priors_cxcy):
    batch = pred_locs.shape[0]
    pad = _P_PAD - _N_PRIORS
    locs_t = jnp.transpose(pred_locs, (0, 2, 1))
    locs_t = jnp.pad(locs_t, ((0, 0), (0, 0), (0, pad)))
    locs_t = locs_t.reshape(batch, 4, _SUB, _LANES)
    scores_t = jnp.transpose(pred_scores, (0, 2, 1))
    scores_t = jnp.pad(scores_t, ((0, 0), (0, 0), (0, pad)))
    scores_t = scores_t.reshape(batch, _N_CLASSES, _SUB, _LANES)
    priors_t = jnp.transpose(priors_cxcy, (1, 0))
    pad_vals = jnp.broadcast_to(
        jnp.array([[10.0], [10.0], [1.0], [1.0]], jnp.float32), (4, pad))
    priors_t = jnp.concatenate([priors_t, pad_vals], axis=1)
    priors_t = priors_t.reshape(4, _SUB, _LANES)
    boxes_r = boxes[:, None]
    labels_r = labels[:, None, None]

    ipp = _IMGS_PER_PROG
    grid = (batch // ipp,)
    out = pl.pallas_call(
        _loss_kernel,
        grid=grid,
        in_specs=[
            pl.BlockSpec((ipp, 1, _N_OBJ, 4), lambda b: (b, 0, 0, 0),
                         memory_space=pltpu.SMEM),
            pl.BlockSpec((ipp, 1, 1, _N_OBJ), lambda b: (b, 0, 0, 0),
                         memory_space=pltpu.SMEM),
            pl.BlockSpec((4, _SUB, _LANES), lambda b: (0, 0, 0)),
            pl.BlockSpec((ipp, 4, _SUB, _LANES), lambda b: (b, 0, 0, 0)),
            pl.BlockSpec((ipp, _N_CLASSES, _SUB, _LANES),
                         lambda b: (b, 0, 0, 0)),
        ],
        out_specs=pl.BlockSpec((1, 1, 1), lambda b: (b, 0, 0),
                               memory_space=pltpu.SMEM),
        out_shape=jax.ShapeDtypeStruct((grid[0], 1, 1), jnp.float32),
        compiler_params=pltpu.CompilerParams(
            dimension_semantics=("parallel",)),
    )(boxes_r, labels_r, priors_t, locs_t, scores_t)
    return jnp.sum(out) / batch


# speedup vs baseline: 1.0080x; 1.0080x over previous
import functools

import jax
import jax.numpy as jnp
from jax.experimental import pallas as pl
from jax.experimental.pallas import tpu as pltpu

_N_PRIORS = 8732
_P_PAD = 9216
_SUB = 8
_LANES = 1152
_N_CLASSES = 21
_N_OBJ = 16
_THRESHOLD = 0.5
_NEG_POS_RATIO = 3
_MINE_ROUNDS = 6
_IMGS_PER_PROG = 4


def _loss_kernel(boxes_ref, labels_ref, priors_ref, locs_ref, scores_ref,
                 out_ref):
    f32 = jnp.float32
    shp = (_SUB, _LANES)

    pcx = priors_ref[0]
    pcy = priors_ref[1]
    pw = priors_ref[2]
    ph = priors_ref[3]
    px0 = pcx - pw * 0.5
    py0 = pcy - ph * 0.5
    px1 = pcx + pw * 0.5
    py1 = pcy + ph * 0.5
    area_p = (px1 - px0) * (py1 - py0)

    iota_p = (jax.lax.broadcasted_iota(jnp.int32, shp, 0) * _LANES
              + jax.lax.broadcasted_iota(jnp.int32, shp, 1))
    valid = iota_p < _N_PRIORS

    def per_image(g):
        best_ov = jnp.full(shp, -1.0, f32)
        lab = jnp.zeros(shp, jnp.int32)
        gx0 = jnp.zeros(shp, f32)
        gy0 = jnp.zeros(shp, f32)
        gx1 = jnp.zeros(shp, f32)
        gy1 = jnp.zeros(shp, f32)
        prior_of_obj = []
        for j in range(_N_OBJ):
            bx0 = boxes_ref[g, 0, j, 0]
            by0 = boxes_ref[g, 0, j, 1]
            bx1 = boxes_ref[g, 0, j, 2]
            by1 = boxes_ref[g, 0, j, 3]
            area_b = (bx1 - bx0) * (by1 - by0)
            wx = jnp.clip(jnp.minimum(bx1, px1) - jnp.maximum(bx0, px0),
                          0.0, None)
            wy = jnp.clip(jnp.minimum(by1, py1) - jnp.maximum(by0, py0),
                          0.0, None)
            inter = wx * wy
            iou = inter / (area_b + area_p - inter)
            mx = jnp.max(iou)
            pj = jnp.min(jnp.where(iou == mx, iota_p, _N_PRIORS))
            prior_of_obj.append(pj)
            upd = iou > best_ov
            best_ov = jnp.where(upd, iou, best_ov)
            lab = jnp.where(upd, labels_ref[g, 0, 0, j], lab)
            gx0 = jnp.where(upd, bx0, gx0)
            gy0 = jnp.where(upd, by0, gy0)
            gx1 = jnp.where(upd, bx1, gx1)
            gy1 = jnp.where(upd, by1, gy1)

        lab = jnp.where(best_ov < _THRESHOLD, 0, lab)
        for j in range(_N_OBJ):
            m = iota_p == prior_of_obj[j]
            lab = jnp.where(m, labels_ref[g, 0, 0, j], lab)
            gx0 = jnp.where(m, boxes_ref[g, 0, j, 0], gx0)
            gy0 = jnp.where(m, boxes_ref[g, 0, j, 1], gy0)
            gx1 = jnp.where(m, boxes_ref[g, 0, j, 2], gx1)
            gy1 = jnp.where(m, boxes_ref[g, 0, j, 3], gy1)
        pos = lab != 0
        posf = pos.astype(f32)
        n_pos = jnp.sum(posf)

        gcx = (gx0 + gx1) * 0.5
        gcy = (gy0 + gy1) * 0.5
        gw = gx1 - gx0
        gh = gy1 - gy0
        t0 = (gcx - pcx) / (pw / 10.0)
        t1 = (gcy - pcy) / (ph / 10.0)
        t2 = jnp.log(gw / pw) * 5.0
        t3 = jnp.log(gh / ph) * 5.0
        sl = jnp.zeros(shp, f32)
        for i, t in enumerate((t0, t1, t2, t3)):
            diff = jnp.abs(locs_ref[g, i] - t)
            sl = sl + jnp.where(diff < 1.0, 0.5 * diff * diff, diff - 0.5)
        loc_loss = jnp.sum(sl * posf)

        m = scores_ref[g, 0]
        for c in range(1, _N_CLASSES):
            m = jnp.maximum(m, scores_ref[g, c])
        ssum = jnp.zeros(shp, f32)
        sal = jnp.zeros(shp, f32)
        for c in range(_N_CLASSES):
            shc = scores_ref[g, c] - m
            ssum = ssum + jnp.exp(shc)
            sal = jnp.where(lab == c, shc, sal)
        conf = jnp.log(ssum) - sal
        conf_pos = jnp.sum(conf * posf)
        neg = jnp.where(pos | jnp.logical_not(valid), 0.0, conf)

        k = n_pos * float(_NEG_POS_RATIO)
        blo = jnp.float32(0.0)
        bhi = jnp.max(neg)
        for _ in range(_MINE_ROUNDS):
            w = (bhi - blo) * 0.125
            cnts = [jnp.sum((neg > (blo + w * float(i))).astype(f32))
                    for i in range(1, 8)]
            s = jnp.float32(0.0)
            for c in cnts:
                s = s + (c > k).astype(f32)
            blo = blo + w * s
            bhi = blo + w
        t = 0.5 * (blo + bhi)
        sel = neg > t
        cnt_t = jnp.sum(sel.astype(f32))
        conf_neg = jnp.sum(jnp.where(sel, neg, 0.0)) + (k - cnt_t) * t

        return (conf_pos + conf_neg + loc_loss) / n_pos

    total = per_image(0)
    for g in range(1, _IMGS_PER_PROG):
        total = total + per_image(g)
    out_ref[0, 0, 0] = total


@jax.jit
def kernel(pred_locs, pred_scores, boxes, labels, priors_cxcy):
    batch = pred_locs.shape[0]
    pad = _P_PAD - _N_PRIORS
    locs_t = jnp.transpose(pred_locs, (0, 2, 1))
    locs_t = jnp.pad(locs_t, ((0, 0), (0, 0), (0, pad)))
    locs_t = locs_t.reshape(batch, 4, _SUB, _LANES)
    scores_t = jnp.transpose(pred_scores, (0, 2, 1))
    scores_t = jnp.pad(scores_t, ((0, 0), (0, 0), (0, pad)))
    scores_t = scores_t.reshape(batch, _N_CLASSES, _SUB, _LANES)
    priors_t = jnp.transpose(priors_cxcy, (1, 0))
    pad_vals = jnp.broadcast_to(
        jnp.array([[10.0], [10.0], [1.0], [1.0]], jnp.float32), (4, pad))
    priors_t = jnp.concatenate([priors_t, pad_vals], axis=1)
    priors_t = priors_t.reshape(4, _SUB, _LANES)
    boxes_r = boxes[:, None]
    labels_r = labels[:, None, None]

    ipp = _IMGS_PER_PROG
    grid = (batch // ipp,)
    out = pl.pallas_call(
        _loss_kernel,
        grid=grid,
        in_specs=[
            pl.BlockSpec((ipp, 1, _N_OBJ, 4), lambda b: (b, 0, 0, 0),
                         memory_space=pltpu.SMEM),
            pl.BlockSpec((ipp, 1, 1, _N_OBJ), lambda b: (b, 0, 0, 0),
                         memory_space=pltpu.SMEM),
            pl.BlockSpec((4, _SUB, _LANES), lambda b: (0, 0, 0)),
            pl.BlockSpec((ipp, 4, _SUB, _LANES), lambda b: (b, 0, 0, 0)),
            pl.BlockSpec((ipp, _N_CLASSES, _SUB, _LANES),
                         lambda b: (b, 0, 0, 0)),
        ],
        out_specs=pl.BlockSpec((1, 1, 1), lambda b: (b, 0, 0),
                               memory_space=pltpu.SMEM),
        out_shape=jax.ShapeDtypeStruct((grid[0], 1, 1), jnp.float32),
        compiler_params=pltpu.CompilerParams(
            dimension_semantics=("parallel",)),
    )(boxes_r, labels_r, priors_t, locs_t, scores_t)
    return jnp.sum(out) / batch
